# Initial kernel scaffold; baseline (speedup 1.0000x reference)
import jax
import jax.numpy as jnp
from jax.experimental import pallas as pl


def kernel(x, edge_index, edge_attr, W1, a_src1, a_dst1, a_edge1, We1, b1, Wrel, brel, Wroot, W2, a_src2, a_dst2, a_edge2, We2, b2, Wg, bg, Wih_f, Whh_f, bih_f, bhh_f, Wih_r, Whh_r, bih_r, bhh_r, Wo, bo):
    raise NotImplementedError("write your pallas kernel here")


# speedup vs baseline: 61.1805x; 61.1805x over previous
import functools
import numpy as np
import jax
import jax.numpy as jnp
from jax import lax
from jax.experimental import pallas as pl
from jax.experimental.pallas import tpu as pltpu, tpu_sc as plsc

F32 = jnp.float32
I32 = jnp.int32
NEG = -1e9
CH = 128
ZCH = 2048


def _lrelu(x):
    return jnp.maximum(x, 0.0) + 0.2 * jnp.minimum(x, 0.0)


def _prep1_body(x_ref, w1_ref, asrc_ref, adst_ref, cepos_ref, ce_ref,
                h_ref, ad_ref, par_ref, mx_sc):
    nb = pl.program_id(1)
    nbk = pl.num_programs(1)
    h = jnp.dot(x_ref[0], w1_ref[...], preferred_element_type=F32)
    h_ref[0] = h
    asrc = jnp.dot(h, asrc_ref[...], preferred_element_type=F32)
    adst = jnp.dot(h, adst_ref[...], preferred_element_type=F32)
    ad_ref[0] = adst
    bmax = jnp.concatenate(
        [asrc.max(0, keepdims=True), adst.max(0, keepdims=True)], axis=1)

    @pl.when(nb == 0)
    def _():
        mx_sc[...] = bmax

    @pl.when(nb > 0)
    def _():
        mx_sc[...] = jnp.maximum(mx_sc[...], bmax)

    @pl.when(nb == nbk - 1)
    def _():
        m = mx_sc[...]
        m1 = _lrelu(m[:, 0:4] + m[:, 4:8] + cepos_ref[...])
        par_ref[0] = jnp.concatenate(
            [m1, ce_ref[...], jnp.zeros((1, 8), F32)], axis=1)


def _mid1_body(acc_ref, b1_ref, wcat_ref, rrep_ref, h1_ref, gr_ref):
    a0 = acc_ref[0, 0]
    a1 = acc_ref[0, 1]
    num = a0[:, 0:16] + a1[:, 0:16]
    den = a0[:, 16:20] + a1[:, 16:20]
    denx = jnp.dot(den, rrep_ref[...], preferred_element_type=F32)
    h1 = jnp.maximum(num / (denx + 1e-16) + b1_ref[...], 0.0)
    h1_ref[0] = h1
    gr_ref[0] = jnp.dot(h1, wcat_ref[...], preferred_element_type=F32)


def _score_body(agg_ref, gr_ref, brel_ref, sc_ref):
    aggs = agg_ref[0].sum(0)
    r1 = gr_ref[0][:, 1:2]
    sc_ref[0] = jnp.tanh(aggs + brel_ref[...] + r1)


def _thr_body(sc_ref, sel_ref, *, kk):
    s = sc_ref[0]
    bits = lax.bitcast_convert_type(s, I32)
    intmin = jnp.int32(-2147483648)
    skey = jnp.where(bits >= 0, bits, intmin - bits)
    cnt0 = jnp.sum((skey >= 0).astype(I32))
    best0 = jnp.where(cnt0 >= kk, jnp.int32(0), intmin)

    def body(i, best):
        bit = jnp.int32(30) - i
        cand = best + jnp.left_shift(jnp.int32(1), bit)
        cnt = jnp.sum((skey >= cand).astype(I32))
        return jnp.where(cnt >= kk, cand, best)

    best = lax.fori_loop(0, 31, body, best0)
    sel_ref[0] = (skey >= best).astype(F32)


def _prep2_body(h1_ref, sc_ref, sel_ref, w2_ref, a2_ref, cep_ref, ce2_ref,
                hp2_ref, ad2_ref, par_ref, mx_sc, *, bn):
    nb = pl.program_id(1)
    nbk = pl.num_programs(1)
    sel = sel_ref[0]
    hp = jnp.dot(h1_ref[0] * sc_ref[0], w2_ref[...],
                 preferred_element_type=F32)
    a2 = jnp.dot(hp, a2_ref[...], preferred_element_type=F32)
    a2m = jnp.where(sel > 0, a2, NEG)
    hp2_ref[0] = jnp.concatenate(
        [hp, a2m[:, 0:1], jnp.zeros((bn, 3), F32)], axis=1)
    ad2_ref[0] = a2m[:, 1:2]
    bmax = a2m.max(0, keepdims=True)

    @pl.when(nb == 0)
    def _():
        mx_sc[...] = bmax

    @pl.when(nb > 0)
    def _():
        mx_sc[...] = jnp.maximum(mx_sc[...], bmax)

    @pl.when(nb == nbk - 1)
    def _():
        m = mx_sc[...]
        m2 = _lrelu(m[:, 0:1] + m[:, 1:2] + cep_ref[...])
        par_ref[0] = jnp.concatenate(
            [m2, ce2_ref[...], jnp.zeros((1, 14), F32)], axis=1)


def _final_body(acc_ref, sel_ref, b2_ref, wg_ref, bg_ref, emb_ref,
                m_sc, s_sc, v_sc, mx_sc):
    nb = pl.program_id(1)
    nbk = pl.num_programs(1)
    a0 = acc_ref[0, 0]
    a1 = acc_ref[0, 1]
    num = a0[:, 0:16] + a1[:, 0:16]
    den = a0[:, 16:17] + a1[:, 16:17]
    h2 = jnp.maximum(num / (den + 1e-16) + b2_ref[...], 0.0)
    sel = sel_ref[0]
    gate = jnp.dot(h2, wg_ref[...], preferred_element_type=F32) + bg_ref[...]
    gmask = jnp.where(sel > 0, gate, -1e30)
    bm = gmask.max(0, keepdims=True).max(1, keepdims=True)

    @pl.when(nb == 0)
    def _():
        m_sc[...] = jnp.full((1, 1), -1e30, F32)
        s_sc[...] = jnp.zeros((1, 1), F32)
        v_sc[...] = jnp.zeros((1, 16), F32)
        mx_sc[...] = jnp.full((1, 16), -1e30, F32)

    mm = jnp.maximum(m_sc[...], bm)
    alpha = jnp.exp(m_sc[...] - mm)
    w = jnp.where(sel > 0, jnp.exp(gate - mm), 0.0)
    s_sc[...] = s_sc[...] * alpha + w.sum(0, keepdims=True)
    v_sc[...] = v_sc[...] * alpha + (w * h2).sum(0, keepdims=True)
    mx_sc[...] = jnp.maximum(
        mx_sc[...], jnp.where(sel > 0, h2, -1e30).max(0, keepdims=True))
    m_sc[...] = mm

    @pl.when(nb == nbk - 1)
    def _():
        att = v_sc[...] / s_sc[...]
        emb_ref[0] = jnp.concatenate([att, mx_sc[...]], axis=1)


def _lstm_body(emb_ref, wihf_ref, whhf_ref, bf_ref, wihr_ref, whhr_ref,
               br_ref, wo_ref, bo_ref, out_ref, *, tt, hid):
    h = jnp.zeros((1, hid), F32)
    c = jnp.zeros((1, hid), F32)
    for t in range(tt):
        xt = emb_ref[t]
        g = (jnp.dot(xt, wihf_ref[...], preferred_element_type=F32)
             + jnp.dot(h, whhf_ref[...], preferred_element_type=F32)
             + bf_ref[...])
        i = jax.nn.sigmoid(g[:, 0:hid])
        f = jax.nn.sigmoid(g[:, hid:2 * hid])
        gg = jnp.tanh(g[:, 2 * hid:3 * hid])
        o = jax.nn.sigmoid(g[:, 3 * hid:4 * hid])
        c = f * c + i * gg
        h = o * jnp.tanh(c)
    xr = emb_ref[tt - 1]
    gr = (jnp.dot(xr, wihr_ref[...], preferred_element_type=F32)
          + br_ref[...])
    ir = jax.nn.sigmoid(gr[:, 0:hid])
    fr = jax.nn.sigmoid(gr[:, hid:2 * hid])
    ggr = jnp.tanh(gr[:, 2 * hid:3 * hid])
    orr = jax.nn.sigmoid(gr[:, 3 * hid:4 * hid])
    cr = ir * ggr
    del fr
    hr = orr * jnp.tanh(cr)
    last = jnp.concatenate([h, hr], axis=1)
    out_ref[...] = jax.nn.sigmoid(
        jnp.dot(last, wo_ref[...], preferred_element_type=F32) + bo_ref[...])


def _bc(vec, j, iota16):
    return jnp.broadcast_to((vec * (iota16 == j).astype(vec.dtype)).sum(), (16,))


def _zero_fill_2d(buf, rows, width):
    z16 = jnp.zeros((16,), F32)
    iota16 = lax.iota(I32, 16)
    n = rows * width

    def body(k, _):
        li = k * 16 + iota16
        plsc.store_scatter(buf, [li // width, li % width], z16)
        return ()

    lax.fori_loop(0, n // 16, body, ())


ZR = 1024


def _choose_tb(t, n, width, ns):
    w8 = ((width + 7) // 8) * 8
    for tb in range(t, 0, -1):
        if t % tb:
            continue
        rows = ((tb * n + ZR - 1) // ZR) * ZR
        if rows * w8 <= 1_350_000:
            return tb
    return 1


def _gat1_sc(srcp, dstp, attrp, htab2, ad1f, par1f, asw, *, t, n, nc, ns, e_real):
    nw = nc * ns
    ep = srcp.shape[0]
    cpw = ep // nw
    nchunk = cpw // CH
    tb = _choose_tb(t, n, 20, ns)
    nphase = t // tb
    rows = ((tb * n + ZR - 1) // ZR) * ZR
    nz = rows // ZR
    mesh = plsc.VectorSubcoreMesh(core_axis_name="c", subcore_axis_name="s",
                                  num_cores=nc, num_subcores=ns)

    @functools.partial(
        pl.kernel, mesh=mesh,
        compiler_params=pltpu.CompilerParams(needs_layout_passes=False, use_tc_tiling_on_sc=False),
        out_type=jax.ShapeDtypeStruct((t * nc * n, 20), F32),
        scratch_types=[
            pltpu.VMEM((CH,), I32), pltpu.VMEM((CH,), I32),
            pltpu.VMEM((CH,), F32),
            pltpu.VMEM((CH,), I32), pltpu.VMEM((CH,), I32),
            pltpu.VMEM((CH,), I32),
            pltpu.VMEM((CH, 16), F32), pltpu.VMEM((CH, 4), F32),
            pltpu.VMEM((CH, 20), F32),
            pltpu.VMEM((t * 16,), F32), pltpu.VMEM((16,), F32),
            pltpu.VMEM((256,), F32), pltpu.VMEM((64,), F32),
            pltpu.VMEM((64,), F32),
            pltpu.VMEM_SHARED((rows, 20), F32),
            pltpu.VMEM((ZR, 20), F32),
            pltpu.SemaphoreType.DMA, pltpu.SemaphoreType.DMA,
            pltpu.SemaphoreType.DMA,
        ],
    )
    def k(src_h, dst_h, attr_h, htab_h, ad_h, par_h, asw_h, out_h,
          sv, dv, av, sgi, dgi, dai, hrow, adrow, stage,
          pbuf, awv, awb, ceb, mb, accum, zv, sem1, sem2, semz):
        iota16 = lax.iota(I32, 16)
        jf = [jnp.full((16,), j, I32) for j in range(20)]
        c = lax.axis_index("c")
        s = lax.axis_index("s")
        wid = c * ns + s
        pltpu.sync_copy(par_h, pbuf)
        pltpu.sync_copy(asw_h, awv)
        awvv = awv[...]
        for j in range(16):
            awb[pl.ds(j * 16, 16)] = _bc(awvv, j, iota16)
        prow0 = pbuf[pl.ds(0, 16)]
        for hd in range(4):
            ceb[pl.ds(hd * 16, 16)] = _bc(prow0, 4 + hd, iota16)
        _zero_fill_2d(zv, ZR, 20)

        for tbi in range(nphase):
            @pl.when(s == 0)
            def _():
                cps = [pltpu.async_copy(zv, accum.at[pl.ds(ci * ZR, ZR)], semz)
                       for ci in range(nz)]
                for cp in cps:
                    cp.wait()
            plsc.subcore_barrier()

            for ti in range(tb):
                t_cur = tbi * tb + ti
                prow = pbuf[pl.ds(t_cur * 16, 16)]
                for hd in range(4):
                    mb[pl.ds(hd * 16, 16)] = _bc(prow, hd, iota16)

                def chunk(ci2, _):
                    base = wid * cpw + ci2 * CH
                    pltpu.sync_copy(src_h.at[pl.ds(base, CH)], sv)
                    pltpu.sync_copy(dst_h.at[pl.ds(base, CH)], dv)
                    pltpu.sync_copy(attr_h.at[pl.ds(base, CH)], av)
                    for g in range(CH // 16):
                        sl = pl.ds(g * 16, 16)
                        sgi[sl] = sv[sl] + t_cur * n
                        dd = dv[sl]
                        dgi[sl] = dd + t_cur * n
                        dai[sl] = dd + ti * n
                    cp1 = pltpu.async_copy(htab_h.at[sgi], hrow, sem1)
                    cp2 = pltpu.async_copy(ad_h.at[dgi], adrow, sem2)
                    cp1.wait()
                    cp2.wait()
                    for g in range(CH // 16):
                        sl = pl.ds(g * 16, 16)
                        eid = iota16 + g * 16
                        attr16 = av[sl]
                        v01 = ((base + g * 16 + iota16) < e_real).astype(F32)
                        hc = [plsc.load_gather(hrow, [eid, jf[j]])
                              for j in range(16)]
                        exs = []
                        for hd in range(4):
                            asrc = (hc[4 * hd] * awb[pl.ds((4 * hd) * 16, 16)]
                                    + hc[4 * hd + 1] * awb[pl.ds((4 * hd + 1) * 16, 16)]
                                    + hc[4 * hd + 2] * awb[pl.ds((4 * hd + 2) * 16, 16)]
                                    + hc[4 * hd + 3] * awb[pl.ds((4 * hd + 3) * 16, 16)])
                            adst = plsc.load_gather(adrow, [eid, jf[hd]])
                            lg = _lrelu(asrc + adst + attr16 * ceb[pl.ds(hd * 16, 16)])
                            ex = jnp.exp(lg - mb[pl.ds(hd * 16, 16)]) * v01
                            exs.append(ex)
                            plsc.store_scatter(stage, [eid, jf[16 + hd]], ex)
                        for j in range(16):
                            plsc.store_scatter(stage, [eid, jf[j]],
                                               hc[j] * exs[j // 4])
                    pltpu.sync_copy(stage, accum.at[dai], add=True)
                    return ()

                lax.fori_loop(0, nchunk, chunk, ())
            plsc.subcore_barrier()

            @pl.when(s == ns - 1)
            def _():
                for ti in range(tb):
                    t_cur2 = tbi * tb + ti
                    pltpu.sync_copy(
                        accum.at[pl.ds(ti * n, n)],
                        out_h.at[pl.ds((t_cur2 * nc + c) * n, n)])
            plsc.subcore_barrier()

    return k(srcp, dstp, attrp, htab2, ad1f, par1f, asw)


def _sag_sc(srcp, dstp, g1f, *, t, n, nc, ns, e_real):
    nw = nc * ns
    ep = srcp.shape[0]
    cpw = ep // nw
    nchunk = cpw // CH
    rows = ((t * n + ZCH - 1) // ZCH) * ZCH
    nz = rows // ZCH
    mesh = plsc.VectorSubcoreMesh(core_axis_name="c", subcore_axis_name="s",
                                  num_cores=nc, num_subcores=ns)

    @functools.partial(
        pl.kernel, mesh=mesh,
        compiler_params=pltpu.CompilerParams(needs_layout_passes=False, use_tc_tiling_on_sc=False),
        out_type=jax.ShapeDtypeStruct((t * nc * n,), F32),
        scratch_types=[
            pltpu.VMEM((CH,), I32), pltpu.VMEM((CH,), I32),
            pltpu.VMEM((CH,), I32), pltpu.VMEM((CH,), F32),
            pltpu.VMEM((n,), F32),
            pltpu.VMEM_SHARED((rows,), F32),
            pltpu.VMEM((ZCH,), F32),
            pltpu.SemaphoreType.DMA,
        ],
    )
    def k(src_h, dst_h, g1_h, out_h, sv, dv, dai, stage, g1t, accum, zv, semz):
        iota16 = lax.iota(I32, 16)
        c = lax.axis_index("c")
        s = lax.axis_index("s")
        wid = c * ns + s

        def zb(kz, _):
            zv[pl.ds(kz * 16, 16)] = jnp.zeros((16,), F32)
            return ()

        lax.fori_loop(0, ZCH // 16, zb, ())
        @pl.when(s == 0)
        def _():
            cps = [pltpu.async_copy(zv, accum.at[pl.ds(ci * ZCH, ZCH)], semz)
                   for ci in range(nz)]
            for cp in cps:
                cp.wait()
        plsc.subcore_barrier()

        for t_cur in range(t):
            pltpu.sync_copy(g1_h.at[pl.ds(t_cur * n, n)], g1t)

            def chunk(ci2, _):
                base = wid * cpw + ci2 * CH
                pltpu.sync_copy(src_h.at[pl.ds(base, CH)], sv)
                pltpu.sync_copy(dst_h.at[pl.ds(base, CH)], dv)
                for g in range(CH // 16):
                    sl = pl.ds(g * 16, 16)
                    vals = plsc.load_gather(g1t, [sv[sl]])
                    v01 = ((base + g * 16 + iota16) < e_real).astype(F32)
                    stage[sl] = vals * v01
                    dai[sl] = dv[sl] + t_cur * n
                pltpu.sync_copy(stage, accum.at[dai], add=True)
                return ()

            lax.fori_loop(0, nchunk, chunk, ())
        plsc.subcore_barrier()

        @pl.when(s == ns - 1)
        def _():
            for t_cur in range(t):
                pltpu.sync_copy(accum.at[pl.ds(t_cur * n, n)], g1t)
                pltpu.sync_copy(g1t, out_h.at[pl.ds((t_cur * nc + c) * n, n)])

    return k(srcp, dstp, g1f)


def _gat2_sc(srcp, dstp, attrp, hp2f, ad2f, par2f, *, t, n, nc, ns, e_real):
    nw = nc * ns
    ep = srcp.shape[0]
    cpw = ep // nw
    nchunk = cpw // CH
    tb = _choose_tb(t, n, 17, ns)
    nphase = t // tb
    rows = ((tb * n + ZR - 1) // ZR) * ZR
    nz = rows // ZR
    mesh = plsc.VectorSubcoreMesh(core_axis_name="c", subcore_axis_name="s",
                                  num_cores=nc, num_subcores=ns)

    @functools.partial(
        pl.kernel, mesh=mesh,
        compiler_params=pltpu.CompilerParams(needs_layout_passes=False, use_tc_tiling_on_sc=False),
        out_type=jax.ShapeDtypeStruct((t * nc * n, 17), F32),
        scratch_types=[
            pltpu.VMEM((CH,), I32), pltpu.VMEM((CH,), I32),
            pltpu.VMEM((CH,), F32),
            pltpu.VMEM((CH,), I32), pltpu.VMEM((CH,), I32),
            pltpu.VMEM((CH,), I32),
            pltpu.VMEM((CH, 20), F32), pltpu.VMEM((CH, 17), F32),
            pltpu.VMEM((t * 16,), F32), pltpu.VMEM((CH, 8), F32),
            pltpu.VMEM_SHARED((rows, 17), F32),
            pltpu.VMEM((ZR, 17), F32),
            pltpu.SemaphoreType.DMA, pltpu.SemaphoreType.DMA,
            pltpu.SemaphoreType.DMA,
        ],
    )
    def k(src_h, dst_h, attr_h, hp_h, ad2_h, par_h, out_h,
          sv, dv, av, sgi, dgi, dai, hrow, stage, pbuf, adrow2, accum, zv,
          sem1, sem2, semz):
        iota16 = lax.iota(I32, 16)
        jf = [jnp.full((16,), j, I32) for j in range(20)]
        c = lax.axis_index("c")
        s = lax.axis_index("s")
        wid = c * ns + s
        pltpu.sync_copy(par_h, pbuf)
        _zero_fill_2d(zv, ZR, 17)

        for tbi in range(nphase):
            @pl.when(s == 0)
            def _():
                cps = [pltpu.async_copy(zv, accum.at[pl.ds(ci * ZR, ZR)], semz)
                       for ci in range(nz)]
                for cp in cps:
                    cp.wait()
            plsc.subcore_barrier()

            for ti in range(tb):
                t_cur = tbi * tb + ti
                prow = pbuf[pl.ds(t_cur * 16, 16)]
                m2b = _bc(prow, 0, iota16)
                ce2b = _bc(prow, 1, iota16)

                def chunk(ci2, _):
                    base = wid * cpw + ci2 * CH
                    pltpu.sync_copy(src_h.at[pl.ds(base, CH)], sv)
                    pltpu.sync_copy(dst_h.at[pl.ds(base, CH)], dv)
                    pltpu.sync_copy(attr_h.at[pl.ds(base, CH)], av)
                    for g in range(CH // 16):
                        sl = pl.ds(g * 16, 16)
                        sgi[sl] = sv[sl] + t_cur * n
                        dgi[sl] = (dv[sl] + t_cur * n) >> 3
                        dai[sl] = dv[sl] + ti * n
                    cp1 = pltpu.async_copy(hp_h.at[sgi], hrow, sem1)
                    cp2 = pltpu.async_copy(ad2_h.at[dgi], adrow2, sem2)
                    cp1.wait()
                    cp2.wait()
                    for g in range(CH // 16):
                        sl = pl.ds(g * 16, 16)
                        eid = iota16 + g * 16
                        attr16 = av[sl]
                        v01 = ((base + g * 16 + iota16) < e_real).astype(F32)
                        asrc = plsc.load_gather(hrow, [eid, jf[16]])
                        adst = plsc.load_gather(adrow2, [eid, dv[sl] & 7])
                        lg = _lrelu(asrc + adst + attr16 * ce2b)
                        ex = jnp.exp(lg - m2b) * v01
                        plsc.store_scatter(stage, [eid, jf[16]], ex)
                        for j in range(16):
                            hcj = plsc.load_gather(hrow, [eid, jf[j]])
                            plsc.store_scatter(stage, [eid, jf[j]], hcj * ex)
                    pltpu.sync_copy(stage, accum.at[dai], add=True)
                    return ()

                lax.fori_loop(0, nchunk, chunk, ())
            plsc.subcore_barrier()

            @pl.when(s == ns - 1)
            def _():
                for ti in range(tb):
                    t_cur2 = tbi * tb + ti
                    pltpu.sync_copy(
                        accum.at[pl.ds(ti * n, n)],
                        out_h.at[pl.ds((t_cur2 * nc + c) * n, n)])
            plsc.subcore_barrier()

    return k(srcp, dstp, attrp, hp2f, ad2f, par2f)


def kernel(x, edge_index, edge_attr, W1, a_src1, a_dst1, a_edge1, We1, b1,
           Wrel, brel, Wroot, W2, a_src2, a_dst2, a_edge2, We2, b2,
           Wg, bg, Wih_f, Whh_f, bih_f, bhh_f, Wih_r, Whh_r, bih_r, bhh_r,
           Wo, bo):
    t, n, f = x.shape
    e = edge_index.shape[1]
    heads, c1 = a_src1.shape
    hid = W2.shape[1]
    kk = int(np.ceil(0.8 * n))
    info = plsc.get_sparse_core_info()
    nc, ns = info.num_cores, info.num_subcores
    nw = nc * ns

    src = edge_index[0].astype(I32)
    dst = edge_index[1].astype(I32)
    attr = edge_attr[:, 0].astype(F32)
    cpw = CH * (-(-e // (nw * CH)))
    ep = nw * cpw
    srcp = jnp.pad(src, (0, ep - e))
    dstp = jnp.pad(dst, (0, ep - e))
    attrp = jnp.pad(attr, (0, ep - e))

    hc1 = heads * c1
    ce1 = (We1.reshape(heads, c1) * a_edge1).sum(-1)
    rows_idx = jnp.arange(hc1)
    Asrc = jnp.zeros((hc1, heads), F32).at[
        rows_idx, rows_idx // c1].set(a_src1.reshape(-1))
    Adst = jnp.zeros((hc1, heads), F32).at[
        rows_idx, rows_idx // c1].set(a_dst1.reshape(-1))
    Rrep = jnp.repeat(jnp.eye(heads, dtype=F32), c1, axis=1)
    Wcat = jnp.concatenate([Wrel, Wroot], axis=1)
    A2 = jnp.concatenate([a_src2.T, a_dst2.T], axis=1)
    ce2 = (We2[0] * a_edge2[0]).sum()
    cepos1 = jnp.maximum(ce1, 0.0)[None]
    cep2 = jnp.maximum(ce2, 0.0).reshape(1, 1)
    ce2v = ce2.reshape(1, 1)
    asw = a_src1.reshape(hc1)

    bn = 5000 if n % 5000 == 0 else n
    nbk = n // bn

    htab, ad1, par1 = pl.pallas_call(
        _prep1_body,
        grid=(t, nbk),
        in_specs=[
            pl.BlockSpec((1, bn, f), lambda i, j: (i, j, 0)),
            pl.BlockSpec((f, hc1), lambda i, j: (0, 0)),
            pl.BlockSpec((hc1, heads), lambda i, j: (0, 0)),
            pl.BlockSpec((hc1, heads), lambda i, j: (0, 0)),
            pl.BlockSpec((1, heads), lambda i, j: (0, 0)),
            pl.BlockSpec((1, heads), lambda i, j: (0, 0)),
        ],
        out_specs=[
            pl.BlockSpec((1, bn, hc1), lambda i, j: (i, j, 0)),
            pl.BlockSpec((1, bn, heads), lambda i, j: (i, j, 0)),
            pl.BlockSpec((1, 1, 16), lambda i, j: (i, 0, 0)),
        ],
        out_shape=[
            jax.ShapeDtypeStruct((t, n, hc1), F32),
            jax.ShapeDtypeStruct((t, n, heads), F32),
            jax.ShapeDtypeStruct((t, 1, 16), F32),
        ],
        scratch_shapes=[pltpu.VMEM((1, 8), F32)],
    )(x, W1, Asrc, Adst, cepos1, ce1[None])

    acc1f = _gat1_sc(srcp, dstp, attrp,
                     htab.reshape(t * n, hc1), ad1.reshape(t * n, heads),
                     par1.reshape(t * 16), asw, t=t, n=n, nc=nc, ns=ns, e_real=e)
    acc1 = acc1f.reshape(t, nc, n, 20)

    h1, gr = pl.pallas_call(
        _mid1_body,
        grid=(t, nbk),
        in_specs=[
            pl.BlockSpec((1, nc, bn, 20), lambda i, j: (i, 0, j, 0)),
            pl.BlockSpec((1, hc1), lambda i, j: (0, 0)),
            pl.BlockSpec((hc1, 2), lambda i, j: (0, 0)),
            pl.BlockSpec((heads, hc1), lambda i, j: (0, 0)),
        ],
        out_specs=[
            pl.BlockSpec((1, bn, hc1), lambda i, j: (i, j, 0)),
            pl.BlockSpec((1, bn, 2), lambda i, j: (i, j, 0)),
        ],
        out_shape=[
            jax.ShapeDtypeStruct((t, n, hc1), F32),
            jax.ShapeDtypeStruct((t, n, 2), F32),
        ],
    )(acc1, b1[None], Wcat, Rrep)

    aggf = _sag_sc(srcp, dstp, gr[..., 0].reshape(t * n),
                   t=t, n=n, nc=nc, ns=ns, e_real=e)
    agg = aggf.reshape(t, nc, n)

    score01 = pl.pallas_call(
        _score_body,
        grid=(t, nbk),
        in_specs=[
            pl.BlockSpec((1, nc, bn, 1), lambda i, j: (i, 0, j, 0)),
            pl.BlockSpec((1, bn, 2), lambda i, j: (i, j, 0)),
            pl.BlockSpec((1, 1), lambda i, j: (0, 0)),
        ],
        out_specs=pl.BlockSpec((1, bn, 1), lambda i, j: (i, j, 0)),
        out_shape=jax.ShapeDtypeStruct((t, n, 1), F32),
    )(agg.reshape(t, nc, n, 1), gr, brel[None])

    selr = pl.pallas_call(
        functools.partial(_thr_body, kk=kk),
        grid=(t,),
        in_specs=[pl.BlockSpec((1, 8, n // 8), lambda i: (i, 0, 0))],
        out_specs=pl.BlockSpec((1, 8, n // 8), lambda i: (i, 0, 0)),
        out_shape=jax.ShapeDtypeStruct((t, 8, n // 8), F32),
    )(score01.reshape(t, 8, n // 8))
    sel = selr.reshape(t, n, 1)

    hp2, ad2, par2 = pl.pallas_call(
        functools.partial(_prep2_body, bn=bn),
        grid=(t, nbk),
        in_specs=[
            pl.BlockSpec((1, bn, hc1), lambda i, j: (i, j, 0)),
            pl.BlockSpec((1, bn, 1), lambda i, j: (i, j, 0)),
            pl.BlockSpec((1, bn, 1), lambda i, j: (i, j, 0)),
            pl.BlockSpec((hc1, hid), lambda i, j: (0, 0)),
            pl.BlockSpec((hid, 2), lambda i, j: (0, 0)),
            pl.BlockSpec((1, 1), lambda i, j: (0, 0)),
            pl.BlockSpec((1, 1), lambda i, j: (0, 0)),
        ],
        out_specs=[
            pl.BlockSpec((1, bn, 20), lambda i, j: (i, j, 0)),
            pl.BlockSpec((1, bn, 1), lambda i, j: (i, j, 0)),
            pl.BlockSpec((1, 1, 16), lambda i, j: (i, 0, 0)),
        ],
        out_shape=[
            jax.ShapeDtypeStruct((t, n, 20), F32),
            jax.ShapeDtypeStruct((t, n, 1), F32),
            jax.ShapeDtypeStruct((t, 1, 16), F32),
        ],
        scratch_shapes=[pltpu.VMEM((1, 2), F32)],
    )(h1, score01, sel, W2, A2, cep2, ce2v)

    acc2f = _gat2_sc(srcp, dstp, attrp, hp2.reshape(t * n, 20),
                     ad2.reshape(t * n // 8, 8), par2.reshape(t * 16),
                     t=t, n=n, nc=nc, ns=ns, e_real=e)
    acc2 = acc2f.reshape(t, nc, n, 17)

    embs = pl.pallas_call(
        _final_body,
        grid=(t, nbk),
        in_specs=[
            pl.BlockSpec((1, nc, bn, 17), lambda i, j: (i, 0, j, 0)),
            pl.BlockSpec((1, bn, 1), lambda i, j: (i, j, 0)),
            pl.BlockSpec((1, hid), lambda i, j: (0, 0)),
            pl.BlockSpec((hid, 1), lambda i, j: (0, 0)),
            pl.BlockSpec((1, 1), lambda i, j: (0, 0)),
        ],
        out_specs=pl.BlockSpec((1, 1, 2 * hid), lambda i, j: (i, 0, 0)),
        out_shape=jax.ShapeDtypeStruct((t, 1, 2 * hid), F32),
        scratch_shapes=[pltpu.VMEM((1, 1), F32), pltpu.VMEM((1, 1), F32),
                        pltpu.VMEM((1, hid), F32), pltpu.VMEM((1, hid), F32)],
    )(acc2, sel, b2[None], Wg, bg[None])

    pred = pl.pallas_call(
        functools.partial(_lstm_body, tt=t, hid=hid),
        in_specs=[
            pl.BlockSpec((t, 1, 2 * hid), lambda: (0, 0, 0)),
            pl.BlockSpec((2 * hid, 4 * hid), lambda: (0, 0)),
            pl.BlockSpec((hid, 4 * hid), lambda: (0, 0)),
            pl.BlockSpec((1, 4 * hid), lambda: (0, 0)),
            pl.BlockSpec((2 * hid, 4 * hid), lambda: (0, 0)),
            pl.BlockSpec((hid, 4 * hid), lambda: (0, 0)),
            pl.BlockSpec((1, 4 * hid), lambda: (0, 0)),
            pl.BlockSpec((2 * hid, 1), lambda: (0, 0)),
            pl.BlockSpec((1, 1), lambda: (0, 0)),
        ],
        out_specs=pl.BlockSpec((1, 1), lambda: (0, 0)),
        out_shape=jax.ShapeDtypeStruct((1, 1), F32),
    )(embs, Wih_f.T, Whh_f.T, (bih_f + bhh_f)[None], Wih_r.T, Whh_r.T,
      (bih_r + bhh_r)[None], Wo.T, bo[None])
    return pred
